# baseline (speedup 1.0000x reference)
import functools

import jax
import jax.numpy as jnp
from jax import lax
from jax.experimental import pallas as pl

NUM_LAYERS = 3
GRAPHS_PER_STEP = 1


def _dot(a, b):
    return lax.dot_general(a, b, (((1,), (0,)), ((), ())),
                           preferred_element_type=jnp.float32)


def _dot_tl(a, b):
    return lax.dot_general(a, b, (((0,), (0,)), ((), ())),
                           preferred_element_type=jnp.float32)


def _dot_tr(a, b):
    return lax.dot_general(a, b, (((1,), (1,)), ((), ())),
                           preferred_element_type=jnp.float32)


def _ggnn_kernel(x_ref, adj_ref, w_ref, wih_ref, whh_ref, bih_ref, bhh_ref,
                 out_ref, *, num_layers, d, n, g):
    bf = jnp.bfloat16
    h = x_ref[:, :, :].reshape(g * n, d)
    A = adj_ref[:, :, :].astype(bf)
    b_ih = bih_ref[:, :]
    b_hh = bhh_ref[:, :]
    wih = wih_ref[:, :].astype(bf)
    whh = whh_ref[:, :].astype(bf)
    wrz = jnp.concatenate([wih[0:2 * d], whh[0:2 * d]], axis=1)
    win = wih[2 * d:3 * d]
    whn = whh[2 * d:3 * d]
    b_rz = b_ih[:, 0:2 * d] + b_hh[:, 0:2 * d]
    b_in = b_ih[:, 2 * d:3 * d]
    b_hn = b_hh[:, 2 * d:3 * d]
    for l in range(num_layers):
        hb = h.astype(bf)
        m = _dot(hb, w_ref[l].astype(bf))
        h_n = _dot_tr(hb, whn) + b_hn
        mb = m.astype(bf)
        aggs = [_dot_tl(A[i], mb[i * n:(i + 1) * n, :]).astype(bf)
                for i in range(g)]
        aggb = jnp.concatenate(aggs, axis=0)
        s_rz = _dot_tr(jnp.concatenate([aggb, hb], axis=1), wrz) + b_rz
        i_n = _dot_tr(aggb, win) + b_in
        r = 0.5 * jnp.tanh(0.5 * s_rz[:, 0:d]) + 0.5
        z = 0.5 * jnp.tanh(0.5 * s_rz[:, d:2 * d]) + 0.5
        nn = jnp.tanh(i_n + r * h_n)
        h = h + (1.0 - z) * (nn - h)
    out_ref[:, :, :] = h.reshape(g, n, d)


def kernel(x, adj, W, W_ih, W_hh, b_ih, b_hh):
    B, N, D = x.shape
    num_layers = W.shape[0]
    g = min(GRAPHS_PER_STEP, B)
    b_ih2 = b_ih.reshape(1, 3 * D)
    b_hh2 = b_hh.reshape(1, 3 * D)
    return pl.pallas_call(
        functools.partial(_ggnn_kernel, num_layers=num_layers, d=D, n=N, g=g),
        grid=(B // g,),
        in_specs=[
            pl.BlockSpec((g, N, D), lambda b: (b, 0, 0)),
            pl.BlockSpec((g, N, N), lambda b: (b, 0, 0)),
            pl.BlockSpec((num_layers, D, D), lambda b: (0, 0, 0)),
            pl.BlockSpec((3 * D, D), lambda b: (0, 0)),
            pl.BlockSpec((3 * D, D), lambda b: (0, 0)),
            pl.BlockSpec((1, 3 * D), lambda b: (0, 0)),
            pl.BlockSpec((1, 3 * D), lambda b: (0, 0)),
        ],
        out_specs=pl.BlockSpec((g, N, D), lambda b: (b, 0, 0)),
        out_shape=jax.ShapeDtypeStruct((B, N, D), jnp.float32),
    )(x, adj, W, W_ih, W_hh, b_ih2, b_hh2)


# speedup vs baseline: 1.3357x; 1.3357x over previous
import functools

import jax
import jax.numpy as jnp
from jax import lax
from jax.experimental import pallas as pl

NUM_LAYERS = 3
GRAPHS_PER_STEP = 2


def _dot(a, b):
    return lax.dot_general(a, b, (((1,), (0,)), ((), ())),
                           preferred_element_type=jnp.float32)


def _dot_tl(a, b):
    return lax.dot_general(a, b, (((0,), (0,)), ((), ())),
                           preferred_element_type=jnp.float32)


def _dot_tr(a, b):
    return lax.dot_general(a, b, (((1,), (1,)), ((), ())),
                           preferred_element_type=jnp.float32)


def _ggnn_kernel(x_ref, adj_ref, w_ref, wih_ref, whh_ref, bih_ref, bhh_ref,
                 out_ref, *, num_layers, d, n, g):
    bf = jnp.bfloat16
    h = x_ref[:, :, :].reshape(g * n, d)
    A = adj_ref[:, :, :].astype(bf)
    b_ih = bih_ref[:, :]
    b_hh = bhh_ref[:, :]
    wih = wih_ref[:, :].astype(bf)
    whh = whh_ref[:, :].astype(bf)
    wrz = jnp.concatenate([wih[0:2 * d], whh[0:2 * d]], axis=1)
    win = wih[2 * d:3 * d]
    whn = whh[2 * d:3 * d]
    b_rz = b_ih[:, 0:2 * d] + b_hh[:, 0:2 * d]
    b_in = b_ih[:, 2 * d:3 * d]
    b_hn = b_hh[:, 2 * d:3 * d]
    for l in range(num_layers):
        hb = h.astype(bf)
        m = _dot(hb, w_ref[l].astype(bf))
        h_n = _dot_tr(hb, whn) + b_hn
        mb = m.astype(bf)
        aggs = [_dot_tl(A[i], mb[i * n:(i + 1) * n, :]).astype(bf)
                for i in range(g)]
        aggb = jnp.concatenate(aggs, axis=0)
        s_rz = _dot_tr(jnp.concatenate([aggb, hb], axis=1), wrz) + b_rz
        i_n = _dot_tr(aggb, win) + b_in
        r = 0.5 * jnp.tanh(0.5 * s_rz[:, 0:d]) + 0.5
        z = 0.5 * jnp.tanh(0.5 * s_rz[:, d:2 * d]) + 0.5
        nn = jnp.tanh(i_n + r * h_n)
        h = h + (1.0 - z) * (nn - h)
    out_ref[:, :, :] = h.reshape(g, n, d)


def kernel(x, adj, W, W_ih, W_hh, b_ih, b_hh):
    B, N, D = x.shape
    num_layers = W.shape[0]
    g = min(GRAPHS_PER_STEP, B)
    b_ih2 = b_ih.reshape(1, 3 * D)
    b_hh2 = b_hh.reshape(1, 3 * D)
    return pl.pallas_call(
        functools.partial(_ggnn_kernel, num_layers=num_layers, d=D, n=N, g=g),
        grid=(B // g,),
        in_specs=[
            pl.BlockSpec((g, N, D), lambda b: (b, 0, 0)),
            pl.BlockSpec((g, N, N), lambda b: (b, 0, 0)),
            pl.BlockSpec((num_layers, D, D), lambda b: (0, 0, 0)),
            pl.BlockSpec((3 * D, D), lambda b: (0, 0)),
            pl.BlockSpec((3 * D, D), lambda b: (0, 0)),
            pl.BlockSpec((1, 3 * D), lambda b: (0, 0)),
            pl.BlockSpec((1, 3 * D), lambda b: (0, 0)),
        ],
        out_specs=pl.BlockSpec((g, N, D), lambda b: (b, 0, 0)),
        out_shape=jax.ShapeDtypeStruct((B, N, D), jnp.float32),
    )(x, adj, W, W_ih, W_hh, b_ih2, b_hh2)
